# Initial kernel scaffold; baseline (speedup 1.0000x reference)
import jax
import jax.numpy as jnp
from jax.experimental import pallas as pl


def kernel(x, edge_index, W1_0, W1_1, b1, gamma, beta, Wmu_0, Wmu_1, b_mu, Wls_0, Wls_1, b_ls):
    raise NotImplementedError("write your pallas kernel here")


# speedup vs baseline: 11.3238x; 11.3238x over previous
import functools

import jax
import jax.numpy as jnp
from jax import lax
from jax.experimental import pallas as pl
from jax.experimental.pallas import tpu as pltpu
from jax.experimental.pallas import tpu_sc as plsc

N = 10000
E = 320000
D = 128

NC = 2
NS = 16
NW = NC * NS
EPT = E // NW
EB = 80
NBLK = EPT // EB
NPAD = 10240
ROWS_PER_TILE = NPAD // NS
ZROWS = 128
DEG_PER_TILE = NPAD // NS

RB = 1000
NB = N // RB

_MESH = dict(core_axis_name="c", subcore_axis_name="s",
             num_cores=NC, num_subcores=NS)
_SC_PARAMS = pltpu.CompilerParams(use_tc_tiling_on_sc=False)


def _deg_body(src_hbm, out_hbm, sidx_v, ones_v, stage_v, acc_sh):
    c = lax.axis_index("c")
    s = lax.axis_index("s")
    w = c * NS + s
    pltpu.sync_copy(src_hbm.at[w], sidx_v)
    for j in range(EB // 16):
        ones_v[pl.ds(j * 16, 16)] = jnp.ones((16,), jnp.float32)
    for j in range(DEG_PER_TILE // 16):
        stage_v[pl.ds(j * 16, 16)] = jnp.zeros((16,), jnp.float32)
    pltpu.sync_copy(stage_v, acc_sh.at[pl.ds(s * DEG_PER_TILE, DEG_PER_TILE)])
    plsc.subcore_barrier()

    def step(j, carry):
        pltpu.sync_copy(ones_v, acc_sh.at[sidx_v.at[j]], add=True)
        return carry

    lax.fori_loop(0, NBLK, step, 0)
    plsc.subcore_barrier()
    pltpu.sync_copy(acc_sh.at[pl.ds(s * DEG_PER_TILE, DEG_PER_TILE)], stage_v)
    pltpu.sync_copy(stage_v, out_hbm.at[c, pl.ds(s * DEG_PER_TILE, DEG_PER_TILE)])


_deg_call = pl.kernel(
    _deg_body,
    out_type=jax.ShapeDtypeStruct((NC, NPAD), jnp.float32),
    mesh=plsc.VectorSubcoreMesh(**_MESH),
    scratch_types=[
        pltpu.VMEM((NBLK, EB), jnp.int32),
        pltpu.VMEM((EB,), jnp.float32),
        pltpu.VMEM((DEG_PER_TILE,), jnp.float32),
        pltpu.VMEM_SHARED((NPAD,), jnp.float32),
    ],
    compiler_params=_SC_PARAMS,
)


DH = D // 2


def _spmm_body(feat0_hbm, feat1_hbm, src_hbm, dst_hbm, out_hbm,
               sidx_v, didx_v, rows_v, zbuf_v, acc_sh, sem):
    c = lax.axis_index("c")
    s = lax.axis_index("s")
    w = c * NS + s
    pltpu.sync_copy(src_hbm.at[w], sidx_v)
    pltpu.sync_copy(dst_hbm.at[w], didx_v)

    for h, feat_hbm in enumerate((feat0_hbm, feat1_hbm)):
        def zrow(i, carry):
            for j in range(DH // 16):
                zbuf_v[i, pl.ds(j * 16, 16)] = jnp.zeros((16,), jnp.float32)
            return carry

        lax.fori_loop(0, ZROWS, zrow, 0)
        for k in range(ROWS_PER_TILE // ZROWS):
            pltpu.sync_copy(
                zbuf_v, acc_sh.at[pl.ds(s * ROWS_PER_TILE + k * ZROWS, ZROWS)])
        plsc.subcore_barrier()

        def step(j, carry):
            pltpu.async_copy(feat_hbm.at[sidx_v.at[j]], rows_v, sem).wait()
            pltpu.sync_copy(rows_v, acc_sh.at[didx_v.at[j]], add=True)
            return carry

        lax.fori_loop(0, NBLK, step, 0)
        plsc.subcore_barrier()
        for k in range(ROWS_PER_TILE // ZROWS):
            r0 = s * ROWS_PER_TILE + k * ZROWS
            pltpu.sync_copy(acc_sh.at[pl.ds(r0, ZROWS)], zbuf_v)
            pltpu.sync_copy(zbuf_v, out_hbm.at[c, h, pl.ds(r0, ZROWS)])


_spmm_call = pl.kernel(
    _spmm_body,
    out_type=jax.ShapeDtypeStruct((NC, 2, NPAD, DH), jnp.float32),
    mesh=plsc.VectorSubcoreMesh(**_MESH),
    scratch_types=[
        pltpu.VMEM((NBLK, EB), jnp.int32),
        pltpu.VMEM((NBLK, EB), jnp.int32),
        pltpu.VMEM((EB, DH), jnp.float32),
        pltpu.VMEM((ZROWS, DH), jnp.float32),
        pltpu.VMEM_SHARED((NPAD, DH), jnp.float32),
        pltpu.SemaphoreType.DMA,
    ],
    compiler_params=_SC_PARAMS,
)


def _dot(a, b):
    return jnp.dot(a, b, preferred_element_type=jnp.float32,
                   precision=lax.Precision.HIGHEST)


def _prep_body(deg_ref, x_ref, dis_ref, xp0_ref, xp1_ref):
    degb = deg_ref[0, 0] + deg_ref[1, 0]
    pos = degb > 0.0
    dis = jnp.where(pos, lax.rsqrt(jnp.where(pos, degb, 1.0)), 0.0)
    dis_ref[...] = dis
    xp = dis * x_ref[...]
    xp0_ref[...] = xp[:, :DH]
    xp1_ref[...] = xp[:, DH:]


def _prep_call(degp, x):
    deg4 = degp[:, :N].reshape(NC, NB, RB, 1)
    return pl.pallas_call(
        _prep_body,
        grid=(NB,),
        in_specs=[
            pl.BlockSpec((NC, 1, RB, 1), lambda i: (0, i, 0, 0)),
            pl.BlockSpec((RB, D), lambda i: (i, 0)),
        ],
        out_specs=[
            pl.BlockSpec((RB, 1), lambda i: (i, 0)),
            pl.BlockSpec((RB, DH), lambda i: (i, 0)),
            pl.BlockSpec((RB, DH), lambda i: (i, 0)),
        ],
        out_shape=[
            jax.ShapeDtypeStruct((N, 1), jnp.float32),
            jax.ShapeDtypeStruct((N, DH), jnp.float32),
            jax.ShapeDtypeStruct((N, DH), jnp.float32),
        ],
    )(deg4, x)


def _mid1_body(x_ref, up_ref, dis_ref, w0_ref, w1_ref, b_ref,
               z_ref, stats_ref, ssum, ssq):
    i = pl.program_id(0)
    uc = up_ref[0] + up_ref[1]
    t = -dis_ref[...] * jnp.concatenate([uc[0], uc[1]], axis=1)
    z = _dot(x_ref[...], w0_ref[...]) + _dot(t, w1_ref[...]) + b_ref[...]
    z_ref[...] = z

    @pl.when(i == 0)
    def _():
        ssum[...] = jnp.zeros_like(ssum)
        ssq[...] = jnp.zeros_like(ssq)

    ssum[...] += jnp.sum(z, axis=0, keepdims=True)
    ssq[...] += jnp.sum(z * z, axis=0, keepdims=True)

    @pl.when(i == NB - 1)
    def _():
        stats_ref[...] = jnp.concatenate([ssum[...], ssq[...]], axis=0)


def _mid1_call(x, up, dis):
    def call(W1_0, W1_1, b1):
        return pl.pallas_call(
            _mid1_body,
            grid=(NB,),
            in_specs=[
                pl.BlockSpec((RB, D), lambda i: (i, 0)),
                pl.BlockSpec((NC, 2, RB, DH), lambda i: (0, 0, i, 0)),
                pl.BlockSpec((RB, 1), lambda i: (i, 0)),
                pl.BlockSpec((D, D), lambda i: (0, 0)),
                pl.BlockSpec((D, D), lambda i: (0, 0)),
                pl.BlockSpec((1, D), lambda i: (0, 0)),
            ],
            out_specs=[
                pl.BlockSpec((RB, D), lambda i: (i, 0)),
                pl.BlockSpec((2, D), lambda i: (0, 0)),
            ],
            out_shape=[
                jax.ShapeDtypeStruct((N, D), jnp.float32),
                jax.ShapeDtypeStruct((2, D), jnp.float32),
            ],
            scratch_shapes=[
                pltpu.VMEM((1, D), jnp.float32),
                pltpu.VMEM((1, D), jnp.float32),
            ],
            compiler_params=pltpu.CompilerParams(
                dimension_semantics=("arbitrary",)),
        )(x, up, dis, W1_0, W1_1, b1)
    return call


def _mid2_body(z_ref, stats_ref, dis_ref, g_ref, bt_ref, wcat_ref,
               hp0_ref, hp1_ref, hw_ref):
    inv_n = jnp.float32(1.0 / N)
    mean = stats_ref[0:1, :] * inv_n
    var = stats_ref[1:2, :] * inv_n - mean * mean
    inv = lax.rsqrt(var + 1e-5)
    h = (z_ref[...] - mean) * inv * g_ref[...] + bt_ref[...]
    h = jnp.maximum(h, 0.0)
    hp = dis_ref[...] * h
    hp0_ref[...] = hp[:, :DH]
    hp1_ref[...] = hp[:, DH:]
    hw_ref[...] = _dot(h, wcat_ref[...])


def _mid2_call(z, stats, dis, gamma2, beta2, Wcat):
    return pl.pallas_call(
        _mid2_body,
        grid=(NB,),
        in_specs=[
            pl.BlockSpec((RB, D), lambda i: (i, 0)),
            pl.BlockSpec((2, D), lambda i: (0, 0)),
            pl.BlockSpec((RB, 1), lambda i: (i, 0)),
            pl.BlockSpec((1, D), lambda i: (0, 0)),
            pl.BlockSpec((1, D), lambda i: (0, 0)),
            pl.BlockSpec((D, D), lambda i: (0, 0)),
        ],
        out_specs=[
            pl.BlockSpec((RB, DH), lambda i: (i, 0)),
            pl.BlockSpec((RB, DH), lambda i: (i, 0)),
            pl.BlockSpec((RB, D), lambda i: (i, 0)),
        ],
        out_shape=[
            jax.ShapeDtypeStruct((N, DH), jnp.float32),
            jax.ShapeDtypeStruct((N, DH), jnp.float32),
            jax.ShapeDtypeStruct((N, D), jnp.float32),
        ],
    )(z, stats, dis, gamma2, beta2, Wcat)


def _fin_body(vp_ref, dis_ref, hw_ref, wcat2_ref, bcat_ref, cat_ref):
    vc = vp_ref[0] + vp_ref[1]
    t = -dis_ref[...] * jnp.concatenate([vc[0], vc[1]], axis=1)
    cat_ref[...] = hw_ref[...] + _dot(t, wcat2_ref[...]) + bcat_ref[...]


def _fin_call(vp, dis, hW, Wcat2, bcat2):
    return pl.pallas_call(
        _fin_body,
        grid=(NB,),
        in_specs=[
            pl.BlockSpec((NC, 2, RB, DH), lambda i: (0, 0, i, 0)),
            pl.BlockSpec((RB, 1), lambda i: (i, 0)),
            pl.BlockSpec((RB, D), lambda i: (i, 0)),
            pl.BlockSpec((D, D), lambda i: (0, 0)),
            pl.BlockSpec((1, D), lambda i: (0, 0)),
        ],
        out_specs=pl.BlockSpec((RB, D), lambda i: (i, 0)),
        out_shape=jax.ShapeDtypeStruct((N, D), jnp.float32),
    )(vp, dis, hW, Wcat2, bcat2)


def kernel(x, edge_index, W1_0, W1_1, b1, gamma, beta,
           Wmu_0, Wmu_1, b_mu, Wls_0, Wls_1, b_ls):
    src3 = edge_index[0].astype(jnp.int32).reshape(NW, NBLK, EB)
    dst3 = edge_index[1].astype(jnp.int32).reshape(NW, NBLK, EB)

    degp = _deg_call(src3)
    dis, xp0, xp1 = _prep_call(degp, x)
    up = _spmm_call(xp0, xp1, src3, dst3)
    z, stats = _mid1_call(x, up, dis)(W1_0, W1_1, b1.reshape(1, D))
    Wcat = jnp.concatenate([Wmu_0, Wls_0], axis=1)
    hp0, hp1, hW = _mid2_call(z, stats, dis, gamma.reshape(1, D),
                              beta.reshape(1, D), Wcat)
    vp = _spmm_call(hp0, hp1, src3, dst3)
    Wcat2 = jnp.concatenate([Wmu_1, Wls_1], axis=1)
    bcat2 = jnp.concatenate([b_mu, b_ls]).reshape(1, D)
    cat = _fin_call(vp, dis, hW, Wcat2, bcat2)
    return cat[:, :D // 2], cat[:, D // 2:]


# speedup vs baseline: 16.0949x; 1.4213x over previous
import functools

import jax
import jax.numpy as jnp
from jax import lax
from jax.experimental import pallas as pl
from jax.experimental.pallas import tpu as pltpu
from jax.experimental.pallas import tpu_sc as plsc

N = 10000
E = 320000
D = 128

NC = 2
NS = 16
NW = NC * NS
EPT = E // NW
EB = 125
NBLK = EPT // EB
NPAIR = NBLK // 2
NPAD = 10240
ROWS_PER_TILE = NPAD // NS
ZROWS = 128
DEG_PER_TILE = NPAD // NS

RB = 1000
NB = N // RB

_MESH = dict(core_axis_name="c", subcore_axis_name="s",
             num_cores=NC, num_subcores=NS)
_SC_PARAMS = pltpu.CompilerParams(use_tc_tiling_on_sc=False)


def _deg_body(src_hbm, out_hbm, sidx_v, ones_v, stage_v, acc_sh):
    c = lax.axis_index("c")
    s = lax.axis_index("s")
    w = c * NS + s
    pltpu.sync_copy(src_hbm.at[w], sidx_v)
    for j in range(128 // 16):
        ones_v[pl.ds(j * 16, 16)] = jnp.ones((16,), jnp.float32)
    for j in range(DEG_PER_TILE // 16):
        stage_v[pl.ds(j * 16, 16)] = jnp.zeros((16,), jnp.float32)
    pltpu.sync_copy(stage_v, acc_sh.at[pl.ds(s * DEG_PER_TILE, DEG_PER_TILE)])
    plsc.subcore_barrier()

    def step(j, carry):
        pltpu.sync_copy(ones_v.at[pl.ds(0, EB)], acc_sh.at[sidx_v.at[j]],
                        add=True)
        return carry

    lax.fori_loop(0, NBLK, step, 0)
    plsc.subcore_barrier()
    pltpu.sync_copy(acc_sh.at[pl.ds(s * DEG_PER_TILE, DEG_PER_TILE)], stage_v)
    pltpu.sync_copy(stage_v, out_hbm.at[c, pl.ds(s * DEG_PER_TILE, DEG_PER_TILE)])


_deg_call = pl.kernel(
    _deg_body,
    out_type=jax.ShapeDtypeStruct((NC, NPAD), jnp.float32),
    mesh=plsc.VectorSubcoreMesh(**_MESH),
    scratch_types=[
        pltpu.VMEM((NBLK, EB), jnp.int32),
        pltpu.VMEM((128,), jnp.float32),
        pltpu.VMEM((DEG_PER_TILE,), jnp.float32),
        pltpu.VMEM_SHARED((NPAD,), jnp.float32),
    ],
    compiler_params=_SC_PARAMS,
)


DH = D // 2


def _spmm_body(feat0_hbm, feat1_hbm, src_hbm, dst_hbm, out_hbm,
               sidx_v, didx_v, rows_v, zbuf_v, acc_sh, gsem0, gsem1):
    c = lax.axis_index("c")
    s = lax.axis_index("s")
    w = c * NS + s
    pltpu.sync_copy(src_hbm.at[w], sidx_v)
    pltpu.sync_copy(dst_hbm.at[w], didx_v)

    for h, feat_hbm in enumerate((feat0_hbm, feat1_hbm)):
        def zrow(i, carry):
            for j in range(DH // 16):
                zbuf_v[i, pl.ds(j * 16, 16)] = jnp.zeros((16,), jnp.float32)
            return carry

        lax.fori_loop(0, ZROWS, zrow, 0)
        for k in range(ROWS_PER_TILE // ZROWS):
            pltpu.sync_copy(
                zbuf_v, acc_sh.at[pl.ds(s * ROWS_PER_TILE + k * ZROWS, ZROWS)])
        plsc.subcore_barrier()

        pltpu.async_copy(feat_hbm.at[sidx_v.at[0]], rows_v.at[0], gsem0)

        def pair(j2, carry):
            j = j2 * 2
            pltpu.make_async_copy(
                feat_hbm.at[sidx_v.at[j]], rows_v.at[0], gsem0).wait()
            pltpu.async_copy(
                feat_hbm.at[sidx_v.at[j + 1]], rows_v.at[1], gsem1)
            pltpu.sync_copy(rows_v.at[0], acc_sh.at[didx_v.at[j]], add=True)
            pltpu.make_async_copy(
                feat_hbm.at[sidx_v.at[j + 1]], rows_v.at[1], gsem1).wait()

            @pl.when(j2 < NPAIR - 1)
            def _():
                pltpu.async_copy(
                    feat_hbm.at[sidx_v.at[j + 2]], rows_v.at[0], gsem0)

            pltpu.sync_copy(rows_v.at[1], acc_sh.at[didx_v.at[j + 1]],
                            add=True)
            return carry

        lax.fori_loop(0, NPAIR, pair, 0)
        plsc.subcore_barrier()
        for k in range(ROWS_PER_TILE // ZROWS):
            r0 = s * ROWS_PER_TILE + k * ZROWS
            pltpu.sync_copy(acc_sh.at[pl.ds(r0, ZROWS)], zbuf_v)
            pltpu.sync_copy(zbuf_v, out_hbm.at[c, h, pl.ds(r0, ZROWS)])


_spmm_call = pl.kernel(
    _spmm_body,
    out_type=jax.ShapeDtypeStruct((NC, 2, NPAD, DH), jnp.float32),
    mesh=plsc.VectorSubcoreMesh(**_MESH),
    scratch_types=[
        pltpu.VMEM((NBLK, EB), jnp.int32),
        pltpu.VMEM((NBLK, EB), jnp.int32),
        pltpu.VMEM((2, EB, DH), jnp.float32),
        pltpu.VMEM((ZROWS, DH), jnp.float32),
        pltpu.VMEM_SHARED((NPAD, DH), jnp.float32),
        pltpu.SemaphoreType.DMA,
        pltpu.SemaphoreType.DMA,
    ],
    compiler_params=_SC_PARAMS,
)


def _dot(a, b):
    return jnp.dot(a, b, preferred_element_type=jnp.float32,
                   precision=lax.Precision.HIGHEST)


def _prep_body(deg_ref, x_ref, dis_ref, xp0_ref, xp1_ref):
    degb = deg_ref[0, 0] + deg_ref[1, 0]
    pos = degb > 0.0
    dis = jnp.where(pos, lax.rsqrt(jnp.where(pos, degb, 1.0)), 0.0)
    dis_ref[...] = dis
    xp = dis * x_ref[...]
    xp0_ref[...] = xp[:, :DH]
    xp1_ref[...] = xp[:, DH:]


def _prep_call(degp, x):
    deg4 = degp[:, :N].reshape(NC, NB, RB, 1)
    return pl.pallas_call(
        _prep_body,
        grid=(NB,),
        in_specs=[
            pl.BlockSpec((NC, 1, RB, 1), lambda i: (0, i, 0, 0)),
            pl.BlockSpec((RB, D), lambda i: (i, 0)),
        ],
        out_specs=[
            pl.BlockSpec((RB, 1), lambda i: (i, 0)),
            pl.BlockSpec((RB, DH), lambda i: (i, 0)),
            pl.BlockSpec((RB, DH), lambda i: (i, 0)),
        ],
        out_shape=[
            jax.ShapeDtypeStruct((N, 1), jnp.float32),
            jax.ShapeDtypeStruct((N, DH), jnp.float32),
            jax.ShapeDtypeStruct((N, DH), jnp.float32),
        ],
    )(deg4, x)


def _mid1_body(x_ref, up_ref, dis_ref, w0_ref, w1_ref, b_ref,
               z_ref, stats_ref, ssum, ssq):
    i = pl.program_id(0)
    uc = up_ref[0] + up_ref[1]
    t = -dis_ref[...] * jnp.concatenate([uc[0], uc[1]], axis=1)
    z = _dot(x_ref[...], w0_ref[...]) + _dot(t, w1_ref[...]) + b_ref[...]
    z_ref[...] = z

    @pl.when(i == 0)
    def _():
        ssum[...] = jnp.zeros_like(ssum)
        ssq[...] = jnp.zeros_like(ssq)

    ssum[...] += jnp.sum(z, axis=0, keepdims=True)
    ssq[...] += jnp.sum(z * z, axis=0, keepdims=True)

    @pl.when(i == NB - 1)
    def _():
        stats_ref[...] = jnp.concatenate([ssum[...], ssq[...]], axis=0)


def _mid1_call(x, up, dis):
    def call(W1_0, W1_1, b1):
        return pl.pallas_call(
            _mid1_body,
            grid=(NB,),
            in_specs=[
                pl.BlockSpec((RB, D), lambda i: (i, 0)),
                pl.BlockSpec((NC, 2, RB, DH), lambda i: (0, 0, i, 0)),
                pl.BlockSpec((RB, 1), lambda i: (i, 0)),
                pl.BlockSpec((D, D), lambda i: (0, 0)),
                pl.BlockSpec((D, D), lambda i: (0, 0)),
                pl.BlockSpec((1, D), lambda i: (0, 0)),
            ],
            out_specs=[
                pl.BlockSpec((RB, D), lambda i: (i, 0)),
                pl.BlockSpec((2, D), lambda i: (0, 0)),
            ],
            out_shape=[
                jax.ShapeDtypeStruct((N, D), jnp.float32),
                jax.ShapeDtypeStruct((2, D), jnp.float32),
            ],
            scratch_shapes=[
                pltpu.VMEM((1, D), jnp.float32),
                pltpu.VMEM((1, D), jnp.float32),
            ],
            compiler_params=pltpu.CompilerParams(
                dimension_semantics=("arbitrary",)),
        )(x, up, dis, W1_0, W1_1, b1)
    return call


def _mid2_body(z_ref, stats_ref, dis_ref, g_ref, bt_ref, wcat_ref,
               hp0_ref, hp1_ref, hw_ref):
    inv_n = jnp.float32(1.0 / N)
    mean = stats_ref[0:1, :] * inv_n
    var = stats_ref[1:2, :] * inv_n - mean * mean
    inv = lax.rsqrt(var + 1e-5)
    h = (z_ref[...] - mean) * inv * g_ref[...] + bt_ref[...]
    h = jnp.maximum(h, 0.0)
    hp = dis_ref[...] * h
    hp0_ref[...] = hp[:, :DH]
    hp1_ref[...] = hp[:, DH:]
    hw_ref[...] = _dot(h, wcat_ref[...])


def _mid2_call(z, stats, dis, gamma2, beta2, Wcat):
    return pl.pallas_call(
        _mid2_body,
        grid=(NB,),
        in_specs=[
            pl.BlockSpec((RB, D), lambda i: (i, 0)),
            pl.BlockSpec((2, D), lambda i: (0, 0)),
            pl.BlockSpec((RB, 1), lambda i: (i, 0)),
            pl.BlockSpec((1, D), lambda i: (0, 0)),
            pl.BlockSpec((1, D), lambda i: (0, 0)),
            pl.BlockSpec((D, D), lambda i: (0, 0)),
        ],
        out_specs=[
            pl.BlockSpec((RB, DH), lambda i: (i, 0)),
            pl.BlockSpec((RB, DH), lambda i: (i, 0)),
            pl.BlockSpec((RB, D), lambda i: (i, 0)),
        ],
        out_shape=[
            jax.ShapeDtypeStruct((N, DH), jnp.float32),
            jax.ShapeDtypeStruct((N, DH), jnp.float32),
            jax.ShapeDtypeStruct((N, D), jnp.float32),
        ],
    )(z, stats, dis, gamma2, beta2, Wcat)


def _fin_body(vp_ref, dis_ref, hw_ref, wcat2_ref, bcat_ref, cat_ref):
    vc = vp_ref[0] + vp_ref[1]
    t = -dis_ref[...] * jnp.concatenate([vc[0], vc[1]], axis=1)
    cat_ref[...] = hw_ref[...] + _dot(t, wcat2_ref[...]) + bcat_ref[...]


def _fin_call(vp, dis, hW, Wcat2, bcat2):
    return pl.pallas_call(
        _fin_body,
        grid=(NB,),
        in_specs=[
            pl.BlockSpec((NC, 2, RB, DH), lambda i: (0, 0, i, 0)),
            pl.BlockSpec((RB, 1), lambda i: (i, 0)),
            pl.BlockSpec((RB, D), lambda i: (i, 0)),
            pl.BlockSpec((D, D), lambda i: (0, 0)),
            pl.BlockSpec((1, D), lambda i: (0, 0)),
        ],
        out_specs=pl.BlockSpec((RB, D), lambda i: (i, 0)),
        out_shape=jax.ShapeDtypeStruct((N, D), jnp.float32),
    )(vp, dis, hW, Wcat2, bcat2)


def kernel(x, edge_index, W1_0, W1_1, b1, gamma, beta,
           Wmu_0, Wmu_1, b_mu, Wls_0, Wls_1, b_ls):
    src3 = edge_index[0].astype(jnp.int32).reshape(NW, NBLK, EB)
    dst3 = edge_index[1].astype(jnp.int32).reshape(NW, NBLK, EB)

    degp = _deg_call(src3)
    dis, xp0, xp1 = _prep_call(degp, x)
    up = _spmm_call(xp0, xp1, src3, dst3)
    z, stats = _mid1_call(x, up, dis)(W1_0, W1_1, b1.reshape(1, D))
    Wcat = jnp.concatenate([Wmu_0, Wls_0], axis=1)
    hp0, hp1, hW = _mid2_call(z, stats, dis, gamma.reshape(1, D),
                              beta.reshape(1, D), Wcat)
    vp = _spmm_call(hp0, hp1, src3, dst3)
    Wcat2 = jnp.concatenate([Wmu_1, Wls_1], axis=1)
    bcat2 = jnp.concatenate([b_mu, b_ls]).reshape(1, D)
    cat = _fin_call(vp, dis, hW, Wcat2, bcat2)
    return cat[:, :D // 2], cat[:, D // 2:]


# speedup vs baseline: 20.5105x; 1.2743x over previous
import functools

import jax
import jax.numpy as jnp
from jax import lax
from jax.experimental import pallas as pl
from jax.experimental.pallas import tpu as pltpu
from jax.experimental.pallas import tpu_sc as plsc

N = 10000
E = 320000
D = 128

NC = 2
NS = 16
NW = NC * NS
EPT = E // NW
EB = 125
NBLK = EPT // EB
NBUF = 4
NQUAD = NBLK // NBUF
NPAD = 10240
ROWS_PER_TILE = NPAD // NS
ZROWS = 128
DEG_PER_TILE = NPAD // NS

RB = 1000
NB = N // RB

_MESH = dict(core_axis_name="c", subcore_axis_name="s",
             num_cores=NC, num_subcores=NS)
_SC_PARAMS = pltpu.CompilerParams(use_tc_tiling_on_sc=False)


def _deg_body(src_hbm, out_hbm, sidx_v, ones_v, stage_v, acc_sh):
    c = lax.axis_index("c")
    s = lax.axis_index("s")
    w = c * NS + s
    pltpu.sync_copy(src_hbm.at[w], sidx_v)
    for j in range(128 // 16):
        ones_v[pl.ds(j * 16, 16)] = jnp.ones((16,), jnp.float32)
    for j in range(DEG_PER_TILE // 16):
        stage_v[pl.ds(j * 16, 16)] = jnp.zeros((16,), jnp.float32)
    pltpu.sync_copy(stage_v, acc_sh.at[pl.ds(s * DEG_PER_TILE, DEG_PER_TILE)])
    plsc.subcore_barrier()

    def step(j, carry):
        pltpu.sync_copy(ones_v.at[pl.ds(0, EB)], acc_sh.at[sidx_v.at[j]],
                        add=True)
        return carry

    lax.fori_loop(0, NBLK, step, 0)
    plsc.subcore_barrier()
    pltpu.sync_copy(acc_sh.at[pl.ds(s * DEG_PER_TILE, DEG_PER_TILE)], stage_v)
    pltpu.sync_copy(stage_v, out_hbm.at[c, pl.ds(s * DEG_PER_TILE, DEG_PER_TILE)])


_deg_call = pl.kernel(
    _deg_body,
    out_type=jax.ShapeDtypeStruct((NC, NPAD), jnp.float32),
    mesh=plsc.VectorSubcoreMesh(**_MESH),
    scratch_types=[
        pltpu.VMEM((NBLK, EB), jnp.int32),
        pltpu.VMEM((128,), jnp.float32),
        pltpu.VMEM((DEG_PER_TILE,), jnp.float32),
        pltpu.VMEM_SHARED((NPAD,), jnp.float32),
    ],
    compiler_params=_SC_PARAMS,
)


DH = D // 2


def _spmm_body(feat0_hbm, feat1_hbm, src_hbm, dst_hbm, out_hbm,
               sidx_v, didx_v, rows_v, zbuf_v, acc_sh, *sems):
    gsem = sems[:NBUF]
    ssem = sems[NBUF:]
    c = lax.axis_index("c")
    s = lax.axis_index("s")
    w = c * NS + s
    pltpu.sync_copy(src_hbm.at[w], sidx_v)
    pltpu.sync_copy(dst_hbm.at[w], didx_v)

    for h, feat_hbm in enumerate((feat0_hbm, feat1_hbm)):
        def zrow(i, carry):
            for j in range(DH // 16):
                zbuf_v[i, pl.ds(j * 16, 16)] = jnp.zeros((16,), jnp.float32)
            return carry

        lax.fori_loop(0, ZROWS, zrow, 0)
        for k in range(ROWS_PER_TILE // ZROWS):
            pltpu.sync_copy(
                zbuf_v, acc_sh.at[pl.ds(s * ROWS_PER_TILE + k * ZROWS, ZROWS)])
        plsc.subcore_barrier()

        for b in range(NBUF):
            pltpu.async_copy(feat_hbm.at[sidx_v.at[b]], rows_v.at[b], gsem[b])

        def quad(j4, carry):
            j = j4 * NBUF
            sdesc = []
            for b in range(NBUF):
                pltpu.make_async_copy(
                    feat_hbm.at[sidx_v.at[j + b]], rows_v.at[b],
                    gsem[b]).wait()
                sdesc.append(pltpu.async_copy(
                    rows_v.at[b], acc_sh.at[didx_v.at[j + b]], ssem[b],
                    add=True))
            for b in range(NBUF):
                sdesc[b].wait()

                @pl.when(j4 < NQUAD - 1)
                def _(b=b):
                    pltpu.async_copy(
                        feat_hbm.at[sidx_v.at[j + NBUF + b]], rows_v.at[b],
                        gsem[b])
            return carry

        lax.fori_loop(0, NQUAD, quad, 0)
        plsc.subcore_barrier()
        for k in range(ROWS_PER_TILE // ZROWS):
            r0 = s * ROWS_PER_TILE + k * ZROWS
            pltpu.sync_copy(acc_sh.at[pl.ds(r0, ZROWS)], zbuf_v)
            pltpu.sync_copy(zbuf_v, out_hbm.at[c, h, pl.ds(r0, ZROWS)])


_spmm_call = pl.kernel(
    _spmm_body,
    out_type=jax.ShapeDtypeStruct((NC, 2, NPAD, DH), jnp.float32),
    mesh=plsc.VectorSubcoreMesh(**_MESH),
    scratch_types=[
        pltpu.VMEM((NBLK, EB), jnp.int32),
        pltpu.VMEM((NBLK, EB), jnp.int32),
        pltpu.VMEM((NBUF, EB, DH), jnp.float32),
        pltpu.VMEM((ZROWS, DH), jnp.float32),
        pltpu.VMEM_SHARED((NPAD, DH), jnp.float32),
    ] + [pltpu.SemaphoreType.DMA] * (2 * NBUF),
    compiler_params=_SC_PARAMS,
)


def _dot(a, b):
    return jnp.dot(a, b, preferred_element_type=jnp.float32,
                   precision=lax.Precision.HIGHEST)


def _prep_body(deg_ref, x_ref, dis_ref, xp0_ref, xp1_ref):
    degb = deg_ref[0, 0] + deg_ref[1, 0]
    pos = degb > 0.0
    dis = jnp.where(pos, lax.rsqrt(jnp.where(pos, degb, 1.0)), 0.0)
    dis_ref[...] = dis
    xp = dis * x_ref[...]
    xp0_ref[...] = xp[:, :DH]
    xp1_ref[...] = xp[:, DH:]


def _prep_call(degp, x):
    deg4 = degp[:, :N].reshape(NC, NB, RB, 1)
    return pl.pallas_call(
        _prep_body,
        grid=(NB,),
        in_specs=[
            pl.BlockSpec((NC, 1, RB, 1), lambda i: (0, i, 0, 0)),
            pl.BlockSpec((RB, D), lambda i: (i, 0)),
        ],
        out_specs=[
            pl.BlockSpec((RB, 1), lambda i: (i, 0)),
            pl.BlockSpec((RB, DH), lambda i: (i, 0)),
            pl.BlockSpec((RB, DH), lambda i: (i, 0)),
        ],
        out_shape=[
            jax.ShapeDtypeStruct((N, 1), jnp.float32),
            jax.ShapeDtypeStruct((N, DH), jnp.float32),
            jax.ShapeDtypeStruct((N, DH), jnp.float32),
        ],
    )(deg4, x)


def _mid1_body(x_ref, up_ref, dis_ref, w0_ref, w1_ref, b_ref,
               z_ref, stats_ref, ssum, ssq):
    i = pl.program_id(0)
    uc = up_ref[0] + up_ref[1]
    t = -dis_ref[...] * jnp.concatenate([uc[0], uc[1]], axis=1)
    z = _dot(x_ref[...], w0_ref[...]) + _dot(t, w1_ref[...]) + b_ref[...]
    z_ref[...] = z

    @pl.when(i == 0)
    def _():
        ssum[...] = jnp.zeros_like(ssum)
        ssq[...] = jnp.zeros_like(ssq)

    ssum[...] += jnp.sum(z, axis=0, keepdims=True)
    ssq[...] += jnp.sum(z * z, axis=0, keepdims=True)

    @pl.when(i == NB - 1)
    def _():
        stats_ref[...] = jnp.concatenate([ssum[...], ssq[...]], axis=0)


def _mid1_call(x, up, dis):
    def call(W1_0, W1_1, b1):
        return pl.pallas_call(
            _mid1_body,
            grid=(NB,),
            in_specs=[
                pl.BlockSpec((RB, D), lambda i: (i, 0)),
                pl.BlockSpec((NC, 2, RB, DH), lambda i: (0, 0, i, 0)),
                pl.BlockSpec((RB, 1), lambda i: (i, 0)),
                pl.BlockSpec((D, D), lambda i: (0, 0)),
                pl.BlockSpec((D, D), lambda i: (0, 0)),
                pl.BlockSpec((1, D), lambda i: (0, 0)),
            ],
            out_specs=[
                pl.BlockSpec((RB, D), lambda i: (i, 0)),
                pl.BlockSpec((2, D), lambda i: (0, 0)),
            ],
            out_shape=[
                jax.ShapeDtypeStruct((N, D), jnp.float32),
                jax.ShapeDtypeStruct((2, D), jnp.float32),
            ],
            scratch_shapes=[
                pltpu.VMEM((1, D), jnp.float32),
                pltpu.VMEM((1, D), jnp.float32),
            ],
            compiler_params=pltpu.CompilerParams(
                dimension_semantics=("arbitrary",)),
        )(x, up, dis, W1_0, W1_1, b1)
    return call


def _mid2_body(z_ref, stats_ref, dis_ref, g_ref, bt_ref, wcat_ref,
               hp0_ref, hp1_ref, hw_ref):
    inv_n = jnp.float32(1.0 / N)
    mean = stats_ref[0:1, :] * inv_n
    var = stats_ref[1:2, :] * inv_n - mean * mean
    inv = lax.rsqrt(var + 1e-5)
    h = (z_ref[...] - mean) * inv * g_ref[...] + bt_ref[...]
    h = jnp.maximum(h, 0.0)
    hp = dis_ref[...] * h
    hp0_ref[...] = hp[:, :DH]
    hp1_ref[...] = hp[:, DH:]
    hw_ref[...] = _dot(h, wcat_ref[...])


def _mid2_call(z, stats, dis, gamma2, beta2, Wcat):
    return pl.pallas_call(
        _mid2_body,
        grid=(NB,),
        in_specs=[
            pl.BlockSpec((RB, D), lambda i: (i, 0)),
            pl.BlockSpec((2, D), lambda i: (0, 0)),
            pl.BlockSpec((RB, 1), lambda i: (i, 0)),
            pl.BlockSpec((1, D), lambda i: (0, 0)),
            pl.BlockSpec((1, D), lambda i: (0, 0)),
            pl.BlockSpec((D, D), lambda i: (0, 0)),
        ],
        out_specs=[
            pl.BlockSpec((RB, DH), lambda i: (i, 0)),
            pl.BlockSpec((RB, DH), lambda i: (i, 0)),
            pl.BlockSpec((RB, D), lambda i: (i, 0)),
        ],
        out_shape=[
            jax.ShapeDtypeStruct((N, DH), jnp.float32),
            jax.ShapeDtypeStruct((N, DH), jnp.float32),
            jax.ShapeDtypeStruct((N, D), jnp.float32),
        ],
    )(z, stats, dis, gamma2, beta2, Wcat)


def _fin_body(vp_ref, dis_ref, hw_ref, wcat2_ref, bcat_ref, cat_ref):
    vc = vp_ref[0] + vp_ref[1]
    t = -dis_ref[...] * jnp.concatenate([vc[0], vc[1]], axis=1)
    cat_ref[...] = hw_ref[...] + _dot(t, wcat2_ref[...]) + bcat_ref[...]


def _fin_call(vp, dis, hW, Wcat2, bcat2):
    return pl.pallas_call(
        _fin_body,
        grid=(NB,),
        in_specs=[
            pl.BlockSpec((NC, 2, RB, DH), lambda i: (0, 0, i, 0)),
            pl.BlockSpec((RB, 1), lambda i: (i, 0)),
            pl.BlockSpec((RB, D), lambda i: (i, 0)),
            pl.BlockSpec((D, D), lambda i: (0, 0)),
            pl.BlockSpec((1, D), lambda i: (0, 0)),
        ],
        out_specs=pl.BlockSpec((RB, D), lambda i: (i, 0)),
        out_shape=jax.ShapeDtypeStruct((N, D), jnp.float32),
    )(vp, dis, hW, Wcat2, bcat2)


def kernel(x, edge_index, W1_0, W1_1, b1, gamma, beta,
           Wmu_0, Wmu_1, b_mu, Wls_0, Wls_1, b_ls):
    src3 = edge_index[0].astype(jnp.int32).reshape(NW, NBLK, EB)
    dst3 = edge_index[1].astype(jnp.int32).reshape(NW, NBLK, EB)

    degp = _deg_call(src3)
    dis, xp0, xp1 = _prep_call(degp, x)
    up = _spmm_call(xp0, xp1, src3, dst3)
    z, stats = _mid1_call(x, up, dis)(W1_0, W1_1, b1.reshape(1, D))
    Wcat = jnp.concatenate([Wmu_0, Wls_0], axis=1)
    hp0, hp1, hW = _mid2_call(z, stats, dis, gamma.reshape(1, D),
                              beta.reshape(1, D), Wcat)
    vp = _spmm_call(hp0, hp1, src3, dst3)
    Wcat2 = jnp.concatenate([Wmu_1, Wls_1], axis=1)
    bcat2 = jnp.concatenate([b_mu, b_ls]).reshape(1, D)
    cat = _fin_call(vp, dis, hW, Wcat2, bcat2)
    return cat[:, :D // 2], cat[:, D // 2:]
